# Initial kernel scaffold; baseline (speedup 1.0000x reference)
import jax
import jax.numpy as jnp
from jax.experimental import pallas as pl


def kernel(fg_scores, reg_scores, anchors, img_size):
    raise NotImplementedError("write your pallas kernel here")


# speedup vs baseline: 93.1198x; 93.1198x over previous
import jax
import jax.numpy as jnp
from jax import lax
from jax.experimental import pallas as pl

_N = 20000
_ROWS = 160
_NP = _ROWS * 128
_K = 300
_TH = 0.7
_OROWS = 3


def _nms_body(s_ref, an_ref, rg_ref, keep_ref, x_ref, y_ref, w_ref, h_ref):
    s0 = s_ref[0]
    xa = an_ref[0, 0]
    ya = an_ref[0, 1]
    wa = an_ref[0, 2]
    ha = an_ref[0, 3]
    ox = rg_ref[0, 0]
    oy = rg_ref[0, 1]
    ow = rg_ref[0, 2]
    oh = rg_ref[0, 3]
    bx = wa * ox + xa
    by = ha * oy + ya
    bw = wa * jnp.exp(ow)
    bh = ha * jnp.exp(oh)
    area = (bw - bx) * (bh - by)

    ridx = (lax.broadcasted_iota(jnp.int32, (_ROWS, 128), 0) * 128
            + lax.broadcasted_iota(jnp.int32, (_ROWS, 128), 1))
    oidx = (lax.broadcasted_iota(jnp.int32, (_OROWS, 128), 0) * 128
            + lax.broadcasted_iota(jnp.int32, (_OROWS, 128), 1))
    neg = jnp.float32(-jnp.inf)

    def cond(st):
        return jnp.logical_not(st[0])

    def body(st):
        (done, i, s, kr, kx, ky, kw, kh, r0, x0, y0, w0, h0) = st
        m = jnp.max(s)
        alive = m > neg
        pick = jnp.min(jnp.where(s == m, ridx, jnp.int32(2**30)))
        cnt = (s0 > m) | ((s0 == m) & (ridx < pick))
        rank = jnp.sum(cnt.astype(jnp.int32))
        pm = ridx == pick
        zf = jnp.float32(0.0)
        px = jnp.sum(jnp.where(pm, bx, zf))
        py = jnp.sum(jnp.where(pm, by, zf))
        pw = jnp.sum(jnp.where(pm, bw, zf))
        ph = jnp.sum(jnp.where(pm, bh, zf))
        pa = jnp.sum(jnp.where(pm, area, zf))
        xx1 = jnp.maximum(px, bx)
        yy1 = jnp.maximum(py, by)
        xx2 = jnp.minimum(pw, bw)
        yy2 = jnp.minimum(ph, bh)
        iw = jnp.maximum(xx2 - xx1, zf)
        ih = jnp.maximum(yy2 - yy1, zf)
        inter = iw * ih
        iou = inter / (pa + area - inter + jnp.float32(1e-9))
        s_new = jnp.where(iou > _TH, neg, s)
        siw = jnp.maximum(pw - px, zf)
        sih = jnp.maximum(ph - py, zf)
        sint = siw * sih
        siou = sint / (pa + pa - sint + jnp.float32(1e-9))
        stuck = jnp.logical_not(siou > _TH)
        wr_r = jnp.where(alive, rank, r0)
        wr_x = jnp.where(alive, px, x0)
        wr_y = jnp.where(alive, py, y0)
        wr_w = jnp.where(alive, pw, w0)
        wr_h = jnp.where(alive, ph, h0)
        single = alive & jnp.logical_not(stuck)
        fmask = (single & (oidx == i)) | (jnp.logical_not(single) & (oidx >= i))
        kr = jnp.where(fmask, wr_r, kr)
        kx = jnp.where(fmask, wr_x, kx)
        ky = jnp.where(fmask, wr_y, ky)
        kw = jnp.where(fmask, wr_w, kw)
        kh = jnp.where(fmask, wr_h, kh)
        i_new = jnp.where(alive, i + 1, i)
        done_new = jnp.logical_not(alive) | stuck | (i_new >= _K)
        first = alive & (i == 0)
        r0n = jnp.where(first, rank, r0)
        x0n = jnp.where(first, px, x0)
        y0n = jnp.where(first, py, y0)
        w0n = jnp.where(first, pw, w0)
        h0n = jnp.where(first, ph, h0)
        s_out = jnp.where(alive, s_new, s)
        return (done_new, i_new, s_out, kr, kx, ky, kw, kh,
                r0n, x0n, y0n, w0n, h0n)

    zi = jnp.zeros((_OROWS, 128), jnp.int32)
    zv = jnp.zeros((_OROWS, 128), jnp.float32)
    st0 = (jnp.bool_(False), jnp.int32(0), s0, zi, zv, zv, zv, zv,
           jnp.int32(0), jnp.float32(0), jnp.float32(0), jnp.float32(0),
           jnp.float32(0))
    st = lax.while_loop(cond, body, st0)
    keep_ref[0] = st[3]
    x_ref[0] = st[4]
    y_ref[0] = st[5]
    w_ref[0] = st[6]
    h_ref[0] = st[7]


def kernel(fg_scores, reg_scores, anchors, img_size):
    del img_size
    B = fg_scores.shape[0]
    pad = _NP - _N
    s_p = jnp.pad(fg_scores, ((0, 0), (0, pad)),
                  constant_values=-jnp.inf).reshape(B, _ROWS, 128)
    an_p = jnp.pad(jnp.transpose(anchors, (0, 2, 1)),
                   ((0, 0), (0, 0), (0, pad))).reshape(B, 4, _ROWS, 128)
    rg_p = jnp.pad(jnp.transpose(reg_scores, (0, 2, 1)),
                   ((0, 0), (0, 0), (0, pad))).reshape(B, 4, _ROWS, 128)
    kr, kx, ky, kw, kh = pl.pallas_call(
        _nms_body,
        grid=(B,),
        in_specs=[
            pl.BlockSpec((1, _ROWS, 128), lambda b: (b, 0, 0)),
            pl.BlockSpec((1, 4, _ROWS, 128), lambda b: (b, 0, 0, 0)),
            pl.BlockSpec((1, 4, _ROWS, 128), lambda b: (b, 0, 0, 0)),
        ],
        out_specs=[pl.BlockSpec((1, _OROWS, 128), lambda b: (b, 0, 0))] * 5,
        out_shape=[jax.ShapeDtypeStruct((B, _OROWS, 128), jnp.int32)]
        + [jax.ShapeDtypeStruct((B, _OROWS, 128), jnp.float32)] * 4,
    )(s_p, an_p, rg_p)
    nf = _OROWS * 128
    keep = kr.reshape(B, nf)[:, :_K]
    sel = jnp.stack([kx.reshape(B, nf)[:, :_K], ky.reshape(B, nf)[:, :_K],
                     kw.reshape(B, nf)[:, :_K], kh.reshape(B, nf)[:, :_K]],
                    axis=-1)
    return sel, keep
